# baseline (speedup 1.0000x reference)
import jax
import jax.numpy as jnp
from jax.experimental import pallas as pl
from jax.experimental.pallas import tpu as pltpu

_N = 512
_D = 64
_H = 64
_KB = 8
_GRID = _N // _KB


def _rep_rows(x, r):
    m, n = x.shape
    return jax.lax.broadcast_in_dim(x, (m, r, n), (0, 2)).reshape(m * r, n)


def _phase0(a_ref, abf_ref, x_ref, xt_ref, w1at_ref, w1b_ref,
            a2_ref, p_ref, qt_ref, s0_ref):
    a = a_ref[...]
    abf = abf_ref[...]
    a2 = jnp.dot(abf, abf, preferred_element_type=jnp.float32)
    a2_ref[...] = a2
    m1 = a * a2
    s1t = jnp.dot(xt_ref[...], m1, preferred_element_type=jnp.float32)
    qt_ref[...] = jnp.dot(w1b_ref[...], s1t, preferred_element_type=jnp.float32)
    p_ref[...] = jnp.dot(x_ref[...], w1at_ref[...], preferred_element_type=jnp.float32)
    deg = jnp.sum(a, axis=0, keepdims=True)
    s0_ref[...] = jnp.dot(deg, x_ref[...], preferred_element_type=jnp.float32)


def _fused(a_ref, abf_ref, a2blk_ref, pblk_ref, qb_ref,
           w2a_ref, w2b_ref, b2c_ref, s0_ref,
           wfc0_ref, wfc1_ref, wfc2_ref, bfc_ref,
           out_ref, t2_scr, ut_acc, s1p_acc, s2_acc, rbt):
    g = pl.program_id(0)
    kh = _KB // 2
    rh = kh * _H

    @pl.when(g == 0)
    def _():
        ut_acc[...] = jnp.zeros_like(ut_acc)
        s1p_acc[...] = jnp.zeros_like(s1p_acc)
        s2_acc[...] = jnp.zeros_like(s2_acc)

    @pl.when(g < _GRID)
    def _():
        ablk_full = a_ref[pl.ds(g * _KB, _KB), :]
        a2_full = a2blk_ref[...]
        uts, s1s = [], []
        for h in range(2):
            ablk = ablk_full[h * kh:(h + 1) * kh, :]
            arep = _rep_rows(ablk, _H)
            a2rep = _rep_rows(a2_full[h * kh:(h + 1) * kh, :], _H)
            gm = a2rep * pblk_ref[pl.ds(h * rh, rh), :] + qb_ref[...]
            f1p = arep * jnp.maximum(gm, 0.0)
            t2p = jnp.dot(f1p.astype(jnp.bfloat16), abf_ref[...],
                          preferred_element_type=jnp.float32) * arep
            t2_scr[pl.ds(g * (_KB * _H) + h * rh, rh), :] = t2p.astype(jnp.bfloat16)
            uts.append(t2p.reshape(kh, _H, _N).sum(axis=0))
            s1s.append(f1p.reshape(kh, _H, _N).sum(axis=0))
        ut_acc[...] += uts[0] + uts[1]
        s1p_acc[...] += s1s[0] + s1s[1]

    @pl.when(g == _GRID)
    def _():
        rbt[...] = jnp.dot(w2b_ref[...], ut_acc[...],
                           preferred_element_type=jnp.float32) + b2c_ref[...]

    @pl.when(g >= _GRID)
    def _():
        gg = g - _GRID
        t2p = t2_scr[pl.ds(gg * (_KB * _H), _KB * _H), :]
        rb = rbt[...]
        acc = s2_acc[...]
        for kk in range(_KB):
            t2kt = t2p[kk * _H:(kk + 1) * _H, :].astype(jnp.float32)
            r2 = jnp.dot(w2a_ref[...], t2kt, preferred_element_type=jnp.float32)
            arow = a_ref[pl.ds(gg * _KB + kk, 1), :]
            acc += arow * jnp.maximum(r2 + rb, 0.0)
        s2_acc[...] = acc

    @pl.when(g == 2 * _GRID - 1)
    def _():
        s1 = jnp.sum(s1p_acc[...], axis=1, keepdims=True)
        s2 = jnp.sum(s2_acc[...], axis=1, keepdims=True)
        tot = (jnp.sum(s0_ref[...] * wfc0_ref[...])
               + jnp.sum(s1 * wfc1_ref[...])
               + jnp.sum(s2 * wfc2_ref[...])
               + bfc_ref[0, 0])
        out_ref[...] = jnp.reshape(tot, (1, 1))


def kernel(X, adj, W1, b1, W2, b2, Wfc, bfc):
    A = adj
    f32 = jnp.float32
    Abf = A.astype(jnp.bfloat16)

    a2, p, qt, s0 = pl.pallas_call(
        _phase0,
        out_shape=[
            jax.ShapeDtypeStruct((_N, _N), f32),
            jax.ShapeDtypeStruct((_N, _H), f32),
            jax.ShapeDtypeStruct((_H, _N), f32),
            jax.ShapeDtypeStruct((1, _D), f32),
        ],
    )(A, Abf, X, X.T, W1[:, :_D].T, W1[:, _D:])

    pflat = p.reshape(_N * _H, 1)
    kh = _KB // 2
    qb = jnp.tile(qt, (kh, 1)) + jnp.tile(b1[:, None], (kh, 1))

    blast = _GRID - 1
    out = pl.pallas_call(
        _fused,
        grid=(2 * _GRID,),
        in_specs=[
            pl.BlockSpec((_N, _N), lambda g: (0, 0)),
            pl.BlockSpec((_N, _N), lambda g: (0, 0)),
            pl.BlockSpec((_KB, _N), lambda g: (jnp.minimum(g, blast), 0)),
            pl.BlockSpec((_KB * _H, 1), lambda g: (jnp.minimum(g, blast), 0)),
            pl.BlockSpec(((_KB // 2) * _H, _N), lambda g: (0, 0)),
            pl.BlockSpec((_H, _H), lambda g: (0, 0)),
            pl.BlockSpec((_H, _H), lambda g: (0, 0)),
            pl.BlockSpec((_H, 1), lambda g: (0, 0)),
            pl.BlockSpec((1, _D), lambda g: (0, 0)),
            pl.BlockSpec((1, _D), lambda g: (0, 0)),
            pl.BlockSpec((_H, 1), lambda g: (0, 0)),
            pl.BlockSpec((_H, 1), lambda g: (0, 0)),
            pl.BlockSpec((1, 1), lambda g: (0, 0)),
        ],
        out_specs=pl.BlockSpec((1, 1), lambda g: (0, 0)),
        out_shape=jax.ShapeDtypeStruct((1, 1), f32),
        scratch_shapes=[
            pltpu.VMEM((_N * _H, _N), jnp.bfloat16),
            pltpu.VMEM((_H, _N), f32),
            pltpu.VMEM((_H, _N), f32),
            pltpu.VMEM((_H, _N), f32),
            pltpu.VMEM((_H, _N), f32),
        ],
        compiler_params=pltpu.CompilerParams(
            dimension_semantics=("arbitrary",)),
    )(A, Abf, a2, pflat, qb,
      W2[:, :_H], W2[:, _H:], b2[:, None], s0,
      Wfc[:, :_D], Wfc[0, _D:_D + _H, None], Wfc[0, _D + _H:, None],
      bfc.reshape(1, 1))

    return out.reshape(1)


# speedup vs baseline: 1.2635x; 1.2635x over previous
import jax
import jax.numpy as jnp
from jax.experimental import pallas as pl
from jax.experimental.pallas import tpu as pltpu

_N = 512
_D = 64
_H = 64
_KB = 16
_GRID = _N // _KB


def _rep_rows(x, r):
    m, n = x.shape
    return jax.lax.broadcast_in_dim(x, (m, r, n), (0, 2)).reshape(m * r, n)


def _phase0(a_ref, abf_ref, x_ref, xt_ref, w1at_ref, w1b_ref,
            a2_ref, p_ref, qt_ref, s0_ref):
    a = a_ref[...]
    abf = abf_ref[...]
    a2 = jnp.dot(abf, abf, preferred_element_type=jnp.float32)
    a2_ref[...] = a2
    m1 = a * a2
    s1t = jnp.dot(xt_ref[...], m1, preferred_element_type=jnp.float32)
    qt_ref[...] = jnp.dot(w1b_ref[...], s1t, preferred_element_type=jnp.float32)
    p_ref[...] = jnp.dot(x_ref[...], w1at_ref[...], preferred_element_type=jnp.float32)
    deg = jnp.sum(a, axis=0, keepdims=True)
    s0_ref[...] = jnp.dot(deg, x_ref[...], preferred_element_type=jnp.float32)


def _fused(a_ref, abf_ref, a2blk_ref, pblk_ref, qb_ref,
           w2a_ref, w2b_ref, b2c_ref, s0_ref,
           wfc0_ref, wfc1_ref, wfc2_ref, bfc_ref,
           out_ref, t2_scr, ut_acc, s1p_acc, s2_acc, rbt):
    g = pl.program_id(0)
    kh = _KB // 2
    rh = kh * _H

    @pl.when(g == 0)
    def _():
        ut_acc[...] = jnp.zeros_like(ut_acc)
        s1p_acc[...] = jnp.zeros_like(s1p_acc)
        s2_acc[...] = jnp.zeros_like(s2_acc)

    @pl.when(g < _GRID)
    def _():
        ablk_full = a_ref[pl.ds(g * _KB, _KB), :]
        a2_full = a2blk_ref[...]
        uts, s1s = [], []
        for h in range(2):
            ablk = ablk_full[h * kh:(h + 1) * kh, :]
            arep = _rep_rows(ablk, _H)
            a2rep = _rep_rows(a2_full[h * kh:(h + 1) * kh, :], _H)
            gm = a2rep * pblk_ref[pl.ds(h * rh, rh), :] + qb_ref[...]
            f1p = arep * jnp.maximum(gm, 0.0)
            t2p = jnp.dot(f1p.astype(jnp.bfloat16), abf_ref[...],
                          preferred_element_type=jnp.float32) * arep
            t2_scr[pl.ds(g * (_KB * _H) + h * rh, rh), :] = t2p.astype(jnp.bfloat16)
            uts.append(t2p.reshape(kh, _H, _N).sum(axis=0))
            s1s.append(f1p.reshape(kh, _H, _N).sum(axis=0))
        ut_acc[...] += uts[0] + uts[1]
        s1p_acc[...] += s1s[0] + s1s[1]

    @pl.when(g == _GRID)
    def _():
        rbt[...] = jnp.dot(w2b_ref[...], ut_acc[...],
                           preferred_element_type=jnp.float32) + b2c_ref[...]

    @pl.when(g >= _GRID)
    def _():
        gg = g - _GRID
        t2p = t2_scr[pl.ds(gg * (_KB * _H), _KB * _H), :]
        rb = rbt[...]
        acc = s2_acc[...]
        for kk in range(_KB):
            t2kt = t2p[kk * _H:(kk + 1) * _H, :].astype(jnp.float32)
            r2 = jnp.dot(w2a_ref[...], t2kt, preferred_element_type=jnp.float32)
            arow = a_ref[pl.ds(gg * _KB + kk, 1), :]
            acc += arow * jnp.maximum(r2 + rb, 0.0)
        s2_acc[...] = acc

    @pl.when(g == 2 * _GRID - 1)
    def _():
        s1 = jnp.sum(s1p_acc[...], axis=1, keepdims=True)
        s2 = jnp.sum(s2_acc[...], axis=1, keepdims=True)
        tot = (jnp.sum(s0_ref[...] * wfc0_ref[...])
               + jnp.sum(s1 * wfc1_ref[...])
               + jnp.sum(s2 * wfc2_ref[...])
               + bfc_ref[0, 0])
        out_ref[...] = jnp.reshape(tot, (1, 1))


def kernel(X, adj, W1, b1, W2, b2, Wfc, bfc):
    A = adj
    f32 = jnp.float32
    Abf = A.astype(jnp.bfloat16)

    a2, p, qt, s0 = pl.pallas_call(
        _phase0,
        out_shape=[
            jax.ShapeDtypeStruct((_N, _N), f32),
            jax.ShapeDtypeStruct((_N, _H), f32),
            jax.ShapeDtypeStruct((_H, _N), f32),
            jax.ShapeDtypeStruct((1, _D), f32),
        ],
    )(A, Abf, X, X.T, W1[:, :_D].T, W1[:, _D:])

    pflat = p.reshape(_N * _H, 1)
    kh = _KB // 2
    qb = jnp.tile(qt, (kh, 1)) + jnp.tile(b1[:, None], (kh, 1))

    blast = _GRID - 1
    out = pl.pallas_call(
        _fused,
        grid=(2 * _GRID,),
        in_specs=[
            pl.BlockSpec((_N, _N), lambda g: (0, 0)),
            pl.BlockSpec((_N, _N), lambda g: (0, 0)),
            pl.BlockSpec((_KB, _N), lambda g: (jnp.minimum(g, blast), 0)),
            pl.BlockSpec((_KB * _H, 1), lambda g: (jnp.minimum(g, blast), 0)),
            pl.BlockSpec(((_KB // 2) * _H, _N), lambda g: (0, 0)),
            pl.BlockSpec((_H, _H), lambda g: (0, 0)),
            pl.BlockSpec((_H, _H), lambda g: (0, 0)),
            pl.BlockSpec((_H, 1), lambda g: (0, 0)),
            pl.BlockSpec((1, _D), lambda g: (0, 0)),
            pl.BlockSpec((1, _D), lambda g: (0, 0)),
            pl.BlockSpec((_H, 1), lambda g: (0, 0)),
            pl.BlockSpec((_H, 1), lambda g: (0, 0)),
            pl.BlockSpec((1, 1), lambda g: (0, 0)),
        ],
        out_specs=pl.BlockSpec((1, 1), lambda g: (0, 0)),
        out_shape=jax.ShapeDtypeStruct((1, 1), f32),
        scratch_shapes=[
            pltpu.VMEM((_N * _H, _N), jnp.bfloat16),
            pltpu.VMEM((_H, _N), f32),
            pltpu.VMEM((_H, _N), f32),
            pltpu.VMEM((_H, _N), f32),
            pltpu.VMEM((_H, _N), f32),
        ],
        compiler_params=pltpu.CompilerParams(
            dimension_semantics=("arbitrary",)),
    )(A, Abf, a2, pflat, qb,
      W2[:, :_H], W2[:, _H:], b2[:, None], s0,
      Wfc[:, :_D], Wfc[0, _D:_D + _H, None], Wfc[0, _D + _H:, None],
      bfc.reshape(1, 1))

    return out.reshape(1)


# speedup vs baseline: 1.4036x; 1.1109x over previous
import jax
import jax.numpy as jnp
from jax.experimental import pallas as pl
from jax.experimental.pallas import tpu as pltpu

_N = 512
_D = 64
_H = 64
_KB = 32
_GRID = _N // _KB


def _rep_rows(x, r):
    m, n = x.shape
    return jax.lax.broadcast_in_dim(x, (m, r, n), (0, 2)).reshape(m * r, n)


def _phase0(a_ref, abf_ref, x_ref, xt_ref, w1at_ref, w1b_ref,
            a2_ref, p_ref, qt_ref, s0_ref):
    a = a_ref[...]
    abf = abf_ref[...]
    a2 = jnp.dot(abf, abf, preferred_element_type=jnp.float32)
    a2_ref[...] = a2
    m1 = a * a2
    s1t = jnp.dot(xt_ref[...], m1, preferred_element_type=jnp.float32)
    qt_ref[...] = jnp.dot(w1b_ref[...], s1t, preferred_element_type=jnp.float32)
    p_ref[...] = jnp.dot(x_ref[...], w1at_ref[...], preferred_element_type=jnp.float32)
    deg = jnp.sum(a, axis=0, keepdims=True)
    s0_ref[...] = jnp.dot(deg, x_ref[...], preferred_element_type=jnp.float32)


def _fused(a_ref, abf_ref, a2blk_ref, pblk_ref, qb_ref,
           w2a_ref, w2b_ref, b2c_ref, s0_ref,
           wfc0_ref, wfc1_ref, wfc2_ref, bfc_ref,
           out_ref, t2_scr, ut_acc, s1p_acc, s2_acc, rbt):
    g = pl.program_id(0)
    kh = _KB // 2
    rh = kh * _H

    @pl.when(g == 0)
    def _():
        ut_acc[...] = jnp.zeros_like(ut_acc)
        s1p_acc[...] = jnp.zeros_like(s1p_acc)
        s2_acc[...] = jnp.zeros_like(s2_acc)

    @pl.when(g < _GRID)
    def _():
        ablk_full = a_ref[pl.ds(g * _KB, _KB), :]
        a2_full = a2blk_ref[...]
        uts, s1s = [], []
        for h in range(2):
            ablk = ablk_full[h * kh:(h + 1) * kh, :]
            arep = _rep_rows(ablk, _H)
            a2rep = _rep_rows(a2_full[h * kh:(h + 1) * kh, :], _H)
            gm = a2rep * pblk_ref[pl.ds(h * rh, rh), :] + qb_ref[...]
            f1p = arep * jnp.maximum(gm, 0.0)
            t2p = jnp.dot(f1p.astype(jnp.bfloat16), abf_ref[...],
                          preferred_element_type=jnp.float32) * arep
            t2_scr[pl.ds(g * (_KB * _H) + h * rh, rh), :] = t2p.astype(jnp.bfloat16)
            uts.append(t2p.reshape(kh, _H, _N).sum(axis=0))
            s1s.append(f1p.reshape(kh, _H, _N).sum(axis=0))
        ut_acc[...] += uts[0] + uts[1]
        s1p_acc[...] += s1s[0] + s1s[1]

    @pl.when(g == _GRID)
    def _():
        rbt[...] = jnp.dot(w2b_ref[...], ut_acc[...],
                           preferred_element_type=jnp.float32) + b2c_ref[...]

    @pl.when(g >= _GRID)
    def _():
        gg = g - _GRID
        t2p = t2_scr[pl.ds(gg * (_KB * _H), _KB * _H), :]
        rb = rbt[...]
        acc = s2_acc[...]
        for kk in range(_KB):
            t2kt = t2p[kk * _H:(kk + 1) * _H, :].astype(jnp.float32)
            r2 = jnp.dot(w2a_ref[...], t2kt, preferred_element_type=jnp.float32)
            arow = a_ref[pl.ds(gg * _KB + kk, 1), :]
            acc += arow * jnp.maximum(r2 + rb, 0.0)
        s2_acc[...] = acc

    @pl.when(g == 2 * _GRID - 1)
    def _():
        s1 = jnp.sum(s1p_acc[...], axis=1, keepdims=True)
        s2 = jnp.sum(s2_acc[...], axis=1, keepdims=True)
        tot = (jnp.sum(s0_ref[...] * wfc0_ref[...])
               + jnp.sum(s1 * wfc1_ref[...])
               + jnp.sum(s2 * wfc2_ref[...])
               + bfc_ref[0, 0])
        out_ref[...] = jnp.reshape(tot, (1, 1))


def kernel(X, adj, W1, b1, W2, b2, Wfc, bfc):
    A = adj
    f32 = jnp.float32
    Abf = A.astype(jnp.bfloat16)

    a2, p, qt, s0 = pl.pallas_call(
        _phase0,
        out_shape=[
            jax.ShapeDtypeStruct((_N, _N), f32),
            jax.ShapeDtypeStruct((_N, _H), f32),
            jax.ShapeDtypeStruct((_H, _N), f32),
            jax.ShapeDtypeStruct((1, _D), f32),
        ],
    )(A, Abf, X, X.T, W1[:, :_D].T, W1[:, _D:])

    pflat = p.reshape(_N * _H, 1)
    kh = _KB // 2
    qb = jnp.tile(qt, (kh, 1)) + jnp.tile(b1[:, None], (kh, 1))

    blast = _GRID - 1
    out = pl.pallas_call(
        _fused,
        grid=(2 * _GRID,),
        in_specs=[
            pl.BlockSpec((_N, _N), lambda g: (0, 0)),
            pl.BlockSpec((_N, _N), lambda g: (0, 0)),
            pl.BlockSpec((_KB, _N), lambda g: (jnp.minimum(g, blast), 0)),
            pl.BlockSpec((_KB * _H, 1), lambda g: (jnp.minimum(g, blast), 0)),
            pl.BlockSpec(((_KB // 2) * _H, _N), lambda g: (0, 0)),
            pl.BlockSpec((_H, _H), lambda g: (0, 0)),
            pl.BlockSpec((_H, _H), lambda g: (0, 0)),
            pl.BlockSpec((_H, 1), lambda g: (0, 0)),
            pl.BlockSpec((1, _D), lambda g: (0, 0)),
            pl.BlockSpec((1, _D), lambda g: (0, 0)),
            pl.BlockSpec((_H, 1), lambda g: (0, 0)),
            pl.BlockSpec((_H, 1), lambda g: (0, 0)),
            pl.BlockSpec((1, 1), lambda g: (0, 0)),
        ],
        out_specs=pl.BlockSpec((1, 1), lambda g: (0, 0)),
        out_shape=jax.ShapeDtypeStruct((1, 1), f32),
        scratch_shapes=[
            pltpu.VMEM((_N * _H, _N), jnp.bfloat16),
            pltpu.VMEM((_H, _N), f32),
            pltpu.VMEM((_H, _N), f32),
            pltpu.VMEM((_H, _N), f32),
            pltpu.VMEM((_H, _N), f32),
        ],
        compiler_params=pltpu.CompilerParams(
            dimension_semantics=("arbitrary",)),
    )(A, Abf, a2, pflat, qb,
      W2[:, :_H], W2[:, _H:], b2[:, None], s0,
      Wfc[:, :_D], Wfc[0, _D:_D + _H, None], Wfc[0, _D + _H:, None],
      bfc.reshape(1, 1))

    return out.reshape(1)


# speedup vs baseline: 1.4176x; 1.0100x over previous
import jax
import jax.numpy as jnp
from jax.experimental import pallas as pl
from jax.experimental.pallas import tpu as pltpu

_N = 512
_D = 64
_H = 64
_KB = 64
_GRID = _N // _KB


def _rep_rows(x, r):
    m, n = x.shape
    return jax.lax.broadcast_in_dim(x, (m, r, n), (0, 2)).reshape(m * r, n)


def _phase0(a_ref, abf_ref, x_ref, xt_ref, w1at_ref, w1b_ref,
            a2_ref, p_ref, qt_ref, s0_ref):
    a = a_ref[...]
    abf = abf_ref[...]
    a2 = jnp.dot(abf, abf, preferred_element_type=jnp.float32)
    a2_ref[...] = a2
    m1 = a * a2
    s1t = jnp.dot(xt_ref[...], m1, preferred_element_type=jnp.float32)
    qt_ref[...] = jnp.dot(w1b_ref[...], s1t, preferred_element_type=jnp.float32)
    p_ref[...] = jnp.dot(x_ref[...], w1at_ref[...], preferred_element_type=jnp.float32)
    deg = jnp.sum(a, axis=0, keepdims=True)
    s0_ref[...] = jnp.dot(deg, x_ref[...], preferred_element_type=jnp.float32)


def _fused(a_ref, abf_ref, a2blk_ref, pblk_ref, qb_ref,
           w2a_ref, w2b_ref, b2c_ref, s0_ref,
           wfc0_ref, wfc1_ref, wfc2_ref, bfc_ref,
           out_ref, t2_scr, ut_acc, s1p_acc, s2_acc, rbt):
    g = pl.program_id(0)
    kh = _KB // 2
    rh = kh * _H

    @pl.when(g == 0)
    def _():
        ut_acc[...] = jnp.zeros_like(ut_acc)
        s1p_acc[...] = jnp.zeros_like(s1p_acc)
        s2_acc[...] = jnp.zeros_like(s2_acc)

    @pl.when(g < _GRID)
    def _():
        ablk_full = a_ref[pl.ds(g * _KB, _KB), :]
        a2_full = a2blk_ref[...]
        uts, s1s = [], []
        for h in range(2):
            ablk = ablk_full[h * kh:(h + 1) * kh, :]
            arep = _rep_rows(ablk, _H)
            a2rep = _rep_rows(a2_full[h * kh:(h + 1) * kh, :], _H)
            gm = a2rep * pblk_ref[pl.ds(h * rh, rh), :] + qb_ref[...]
            f1p = arep * jnp.maximum(gm, 0.0)
            t2p = jnp.dot(f1p.astype(jnp.bfloat16), abf_ref[...],
                          preferred_element_type=jnp.float32) * arep
            t2_scr[pl.ds(g * (_KB * _H) + h * rh, rh), :] = t2p.astype(jnp.bfloat16)
            uts.append(t2p.reshape(kh, _H, _N).sum(axis=0))
            s1s.append(f1p.reshape(kh, _H, _N).sum(axis=0))
        ut_acc[...] += uts[0] + uts[1]
        s1p_acc[...] += s1s[0] + s1s[1]

    @pl.when(g == _GRID)
    def _():
        rbt[...] = jnp.dot(w2b_ref[...], ut_acc[...],
                           preferred_element_type=jnp.float32) + b2c_ref[...]

    @pl.when(g >= _GRID)
    def _():
        gg = g - _GRID
        t2p = t2_scr[pl.ds(gg * (_KB * _H), _KB * _H), :]
        rb = rbt[...]
        acc = s2_acc[...]
        for kk in range(_KB):
            t2kt = t2p[kk * _H:(kk + 1) * _H, :].astype(jnp.float32)
            r2 = jnp.dot(w2a_ref[...], t2kt, preferred_element_type=jnp.float32)
            arow = a_ref[pl.ds(gg * _KB + kk, 1), :]
            acc += arow * jnp.maximum(r2 + rb, 0.0)
        s2_acc[...] = acc

    @pl.when(g == 2 * _GRID - 1)
    def _():
        s1 = jnp.sum(s1p_acc[...], axis=1, keepdims=True)
        s2 = jnp.sum(s2_acc[...], axis=1, keepdims=True)
        tot = (jnp.sum(s0_ref[...] * wfc0_ref[...])
               + jnp.sum(s1 * wfc1_ref[...])
               + jnp.sum(s2 * wfc2_ref[...])
               + bfc_ref[0, 0])
        out_ref[...] = jnp.reshape(tot, (1, 1))


def kernel(X, adj, W1, b1, W2, b2, Wfc, bfc):
    A = adj
    f32 = jnp.float32
    Abf = A.astype(jnp.bfloat16)

    a2, p, qt, s0 = pl.pallas_call(
        _phase0,
        out_shape=[
            jax.ShapeDtypeStruct((_N, _N), f32),
            jax.ShapeDtypeStruct((_N, _H), f32),
            jax.ShapeDtypeStruct((_H, _N), f32),
            jax.ShapeDtypeStruct((1, _D), f32),
        ],
    )(A, Abf, X, X.T, W1[:, :_D].T, W1[:, _D:])

    pflat = p.reshape(_N * _H, 1)
    kh = _KB // 2
    qb = jnp.tile(qt, (kh, 1)) + jnp.tile(b1[:, None], (kh, 1))

    blast = _GRID - 1
    out = pl.pallas_call(
        _fused,
        grid=(2 * _GRID,),
        in_specs=[
            pl.BlockSpec((_N, _N), lambda g: (0, 0)),
            pl.BlockSpec((_N, _N), lambda g: (0, 0)),
            pl.BlockSpec((_KB, _N), lambda g: (jnp.minimum(g, blast), 0)),
            pl.BlockSpec((_KB * _H, 1), lambda g: (jnp.minimum(g, blast), 0)),
            pl.BlockSpec(((_KB // 2) * _H, _N), lambda g: (0, 0)),
            pl.BlockSpec((_H, _H), lambda g: (0, 0)),
            pl.BlockSpec((_H, _H), lambda g: (0, 0)),
            pl.BlockSpec((_H, 1), lambda g: (0, 0)),
            pl.BlockSpec((1, _D), lambda g: (0, 0)),
            pl.BlockSpec((1, _D), lambda g: (0, 0)),
            pl.BlockSpec((_H, 1), lambda g: (0, 0)),
            pl.BlockSpec((_H, 1), lambda g: (0, 0)),
            pl.BlockSpec((1, 1), lambda g: (0, 0)),
        ],
        out_specs=pl.BlockSpec((1, 1), lambda g: (0, 0)),
        out_shape=jax.ShapeDtypeStruct((1, 1), f32),
        scratch_shapes=[
            pltpu.VMEM((_N * _H, _N), jnp.bfloat16),
            pltpu.VMEM((_H, _N), f32),
            pltpu.VMEM((_H, _N), f32),
            pltpu.VMEM((_H, _N), f32),
            pltpu.VMEM((_H, _N), f32),
        ],
        compiler_params=pltpu.CompilerParams(
            dimension_semantics=("arbitrary",)),
    )(A, Abf, a2, pflat, qb,
      W2[:, :_H], W2[:, _H:], b2[:, None], s0,
      Wfc[:, :_D], Wfc[0, _D:_D + _H, None], Wfc[0, _D + _H:, None],
      bfc.reshape(1, 1))

    return out.reshape(1)
